# baseline (speedup 1.0000x reference)
import functools

import jax
import jax.numpy as jnp
from jax import lax
from jax.experimental import pallas as pl
from jax.experimental.pallas import tpu as pltpu
from jax.experimental.pallas import tpu_sc as plsc

N = 10000
E = 320000
D_IN = 128
D_OUT = 128
HALF = D_OUT // 2

NC = 2
NS = 16
LANES = 16

CHUNK = 128
EDGES_PER_TILE = 20480
EPAD = NS * EDGES_PER_TILE
PHASES = 2
EDGES_PER_PHASE = EDGES_PER_TILE // PHASES
CHUNKS_PER_PHASE = EDGES_PER_PHASE // CHUNK
NBUF = 4
NITER = CHUNKS_PER_PHASE // NBUF
AHEAD = 2
ROWS_PER_TILE = 632
NPAD = NS * ROWS_PER_TILE
SUBSTRIPE = 158


def _lane_splat(v, l):
    idx = jnp.full((LANES, 1), l, jnp.int32)
    dnums = lax.GatherDimensionNumbers(
        offset_dims=(), collapsed_slice_dims=(0,), start_index_map=(0,))
    return lax.gather(v, idx, dnums, (1,),
                      mode=lax.GatherScatterMode.PROMISE_IN_BOUNDS)


def _mm_body(x_ref, w_ref, o_ref):
    o_ref[...] = jnp.dot(x_ref[...], w_ref[0], preferred_element_type=jnp.float32)


def _matmul_split(x, W2):
    grid = (2, 10)
    return pl.pallas_call(
        _mm_body,
        grid=grid,
        in_specs=[
            pl.BlockSpec((N // 10, D_IN), lambda c, i: (i, 0)),
            pl.BlockSpec((1, D_IN, HALF), lambda c, i: (c, 0, 0)),
        ],
        out_specs=pl.BlockSpec((N // 10, HALF), lambda c, i: (c * 10 + i, 0)),
        out_shape=jax.ShapeDtypeStruct((2 * N, HALF), jnp.float32),
    )(x, W2)


def _sc_body(xw_hbm, row_hbm, col_hbm, val_hbm, out_hbm,
             colall, rowbuf2, valbuf2, rows, outbuf, acc, gsem, ssem):
    c = lax.axis_index("c")
    s = lax.axis_index("s")

    zero16 = jnp.zeros((LANES,), jnp.float32)

    def zero_body(i, _):
        for j in range(HALF // LANES):
            outbuf[i, pl.ds(j * LANES, LANES)] = zero16
        return 0

    lax.fori_loop(0, SUBSTRIPE, zero_body, 0)
    for t in range(ROWS_PER_TILE // SUBSTRIPE):
        pltpu.sync_copy(
            outbuf,
            acc.at[pl.ds(s * ROWS_PER_TILE + t * SUBSTRIPE, SUBSTRIPE)])

    col_off = c * N

    def issue_gather(g, k):
        pltpu.make_async_copy(
            xw_hbm.at[colall.at[pl.ds(g * CHUNK, CHUNK)]], rows[k], gsem[k]
        ).start()

    def wait_gather(k):
        pltpu.make_async_copy(
            xw_hbm.at[colall.at[pl.ds(0, CHUNK)]], rows[k], gsem[k]
        ).wait()

    def start_scatter(g, k):
        pltpu.make_async_copy(
            rows[k], acc.at[rowbuf2.at[g]], ssem[k]
        ).start(add=True)

    def wait_scatter(k):
        pltpu.make_async_copy(
            rows[k], acc.at[rowbuf2.at[0]], ssem[k]
        ).wait()

    def scale(g, k):
        for g2 in range(CHUNK // LANES):
            vvals = valbuf2[g, pl.ds(g2 * LANES, LANES)]
            for l in range(LANES):
                vsplat = _lane_splat(vvals, l)
                i = g2 * LANES + l
                for j in range(HALF // LANES):
                    sl = pl.ds(j * LANES, LANES)
                    rows[k][i, sl] = rows[k][i, sl] * vsplat

    def phase_body(p, _):
        ebase = s * EDGES_PER_TILE + p * EDGES_PER_PHASE
        cbase = s * (PHASES * CHUNKS_PER_PHASE) + p * CHUNKS_PER_PHASE
        pltpu.sync_copy(col_hbm.at[pl.ds(ebase, EDGES_PER_PHASE)], colall)
        pltpu.sync_copy(row_hbm.at[pl.ds(cbase, CHUNKS_PER_PHASE)], rowbuf2)
        pltpu.sync_copy(val_hbm.at[pl.ds(cbase, CHUNKS_PER_PHASE)], valbuf2)

        def off_body(k, _):
            sl = pl.ds(k * LANES, LANES)
            colall[sl] = colall[sl] + col_off
            return 0

        lax.fori_loop(0, EDGES_PER_PHASE // LANES, off_body, 0)

        for k0 in range(AHEAD):
            issue_gather(k0, k0)

        def loop_body(i, _):
            for k in range(NBUF):
                g = i * NBUF + k
                k_nx = (k + AHEAD) % NBUF
                if k < NBUF - AHEAD:
                    @pl.when(i > 0)
                    def _():
                        wait_scatter(k_nx)

                    issue_gather(g + AHEAD, k_nx)
                else:
                    @pl.when(i < NITER - 1)
                    def _():
                        wait_scatter(k_nx)
                        issue_gather(g + AHEAD, k_nx)

                wait_gather(k)
                scale(g, k)
                start_scatter(g, k)
            return 0

        lax.fori_loop(0, NITER, loop_body, 0)
        for k in range(NBUF):
            wait_scatter(k)
        return 0

    lax.fori_loop(0, PHASES, phase_body, 0)
    plsc.subcore_barrier()

    def fin_body(t, _):
        rbase = s * ROWS_PER_TILE + t * SUBSTRIPE
        pltpu.sync_copy(acc.at[pl.ds(rbase, SUBSTRIPE)], outbuf)

        def relu_body(i, _):
            for j in range(HALF // LANES):
                sl = pl.ds(j * LANES, LANES)
                outbuf[i, sl] = jnp.maximum(outbuf[i, sl], 0.0)
            return 0

        lax.fori_loop(0, SUBSTRIPE, relu_body, 0)
        pltpu.sync_copy(outbuf, out_hbm.at[pl.ds(c * NPAD + rbase, SUBSTRIPE)])
        return 0

    lax.fori_loop(0, ROWS_PER_TILE // SUBSTRIPE, fin_body, 0)


@functools.partial(
    pl.kernel,
    out_type=jax.ShapeDtypeStruct((2 * NPAD, HALF), jnp.float32),
    mesh=plsc.VectorSubcoreMesh(core_axis_name="c", subcore_axis_name="s"),
    scratch_types=[
        pltpu.VMEM((EDGES_PER_PHASE,), jnp.int32),
        pltpu.VMEM((CHUNKS_PER_PHASE, CHUNK), jnp.int32),
        pltpu.VMEM((CHUNKS_PER_PHASE, CHUNK), jnp.float32),
        [pltpu.VMEM((CHUNK, HALF), jnp.float32) for _ in range(NBUF)],
        pltpu.VMEM((SUBSTRIPE, HALF), jnp.float32),
        pltpu.VMEM_SHARED((NPAD, HALF), jnp.float32),
        [pltpu.SemaphoreType.DMA for _ in range(NBUF)],
        [pltpu.SemaphoreType.DMA for _ in range(NBUF)],
    ],
    compiler_params=pltpu.CompilerParams(use_tc_tiling_on_sc=False),
)
def _sc_aggregate(xw_hbm, row_hbm, col_hbm, val_hbm, out_hbm,
                  colall, rowbuf2, valbuf2, rows, outbuf, acc, gsem, ssem):
    _sc_body(xw_hbm, row_hbm, col_hbm, val_hbm, out_hbm,
             colall, rowbuf2, valbuf2, rows, outbuf, acc, gsem, ssem)


@jax.jit
def kernel(x, adj_row, adj_col, adj_val, W):
    W2 = W.reshape(D_IN, 2, HALF).transpose(1, 0, 2)
    xw = _matmul_split(x, W2)
    npad_e = EPAD - E
    colp = jnp.concatenate(
        [adj_col, jnp.zeros((npad_e,), jnp.int32)])
    valp = jnp.concatenate(
        [adj_val, jnp.zeros((npad_e,), jnp.float32)])
    rowp = jnp.concatenate(
        [adj_row, jnp.full((npad_e,), NPAD - 1, jnp.int32)])
    row2 = rowp.reshape(EPAD // CHUNK, CHUNK)
    val2 = valp.reshape(EPAD // CHUNK, CHUNK)
    out2 = _sc_aggregate(xw, row2, colp, val2)
    return jnp.concatenate([out2[:N], out2[NPAD:NPAD + N]], axis=1)


# speedup vs baseline: 1.9228x; 1.9228x over previous
import functools

import jax
import jax.numpy as jnp
from jax import lax
from jax.experimental import pallas as pl
from jax.experimental.pallas import tpu as pltpu
from jax.experimental.pallas import tpu_sc as plsc

N = 10000
E = 320000
D_IN = 128
D_OUT = 128
HALF = D_OUT // 2

NC = 2
NS = 16
LANES = 16

CHUNK = 80
EDGES_PER_TILE = E // NS
PHASES = 2
EDGES_PER_PHASE = EDGES_PER_TILE // PHASES
CHUNKS_PER_PHASE = EDGES_PER_PHASE // CHUNK
NBUF = 5
NITER = CHUNKS_PER_PHASE // NBUF
AHEAD = 2
ROWS_PER_TILE = 632
NPAD = NS * ROWS_PER_TILE
SUBSTRIPE = 158


def _lane_splat(v, l):
    idx = jnp.full((LANES, 1), l, jnp.int32)
    dnums = lax.GatherDimensionNumbers(
        offset_dims=(), collapsed_slice_dims=(0,), start_index_map=(0,))
    return lax.gather(v, idx, dnums, (1,),
                      mode=lax.GatherScatterMode.PROMISE_IN_BOUNDS)


def _mm_body(x_ref, w_ref, o_ref):
    o_ref[...] = jnp.dot(x_ref[...], w_ref[0], preferred_element_type=jnp.float32)


def _matmul_split(x, W2):
    grid = (2, 10)
    return pl.pallas_call(
        _mm_body,
        grid=grid,
        in_specs=[
            pl.BlockSpec((N // 10, D_IN), lambda c, i: (i, 0)),
            pl.BlockSpec((1, D_IN, HALF), lambda c, i: (c, 0, 0)),
        ],
        out_specs=pl.BlockSpec((N // 10, HALF), lambda c, i: (c * 10 + i, 0)),
        out_shape=jax.ShapeDtypeStruct((2 * N, HALF), jnp.float32),
    )(x, W2)


def _sc_body(xw_hbm, row_hbm, col_hbm, val_hbm, out_hbm,
             colall, rowbuf2, valbuf2, rows, outbuf, acc, gsem, ssem):
    c = lax.axis_index("c")
    s = lax.axis_index("s")

    zero16 = jnp.zeros((LANES,), jnp.float32)

    def zero_body(i, _):
        for j in range(HALF // LANES):
            outbuf[i, pl.ds(j * LANES, LANES)] = zero16
        return 0

    lax.fori_loop(0, SUBSTRIPE, zero_body, 0)
    for t in range(ROWS_PER_TILE // SUBSTRIPE):
        pltpu.sync_copy(
            outbuf,
            acc.at[pl.ds(s * ROWS_PER_TILE + t * SUBSTRIPE, SUBSTRIPE)])

    col_off = c * N

    def issue_gather(g, k):
        pltpu.make_async_copy(
            xw_hbm.at[colall.at[pl.ds(g * CHUNK, CHUNK)]], rows[k], gsem[k]
        ).start()

    def wait_gather(k):
        pltpu.make_async_copy(
            xw_hbm.at[colall.at[pl.ds(0, CHUNK)]], rows[k], gsem[k]
        ).wait()

    def start_scatter(g, k):
        pltpu.make_async_copy(
            rows[k], acc.at[rowbuf2.at[g]], ssem[k]
        ).start(add=True)

    def wait_scatter(k):
        pltpu.make_async_copy(
            rows[k], acc.at[rowbuf2.at[0]], ssem[k]
        ).wait()

    def scale(g, k):
        for g2 in range(CHUNK // LANES):
            vvals = valbuf2[g, pl.ds(g2 * LANES, LANES)]
            for l in range(LANES):
                vsplat = _lane_splat(vvals, l)
                i = g2 * LANES + l
                for j in range(HALF // LANES):
                    sl = pl.ds(j * LANES, LANES)
                    rows[k][i, sl] = rows[k][i, sl] * vsplat

    def phase_body(p, _):
        ebase = s * EDGES_PER_TILE + p * EDGES_PER_PHASE
        cbase = s * (PHASES * CHUNKS_PER_PHASE) + p * CHUNKS_PER_PHASE
        pltpu.sync_copy(col_hbm.at[pl.ds(ebase, EDGES_PER_PHASE)], colall)
        pltpu.sync_copy(row_hbm.at[pl.ds(cbase, CHUNKS_PER_PHASE)], rowbuf2)
        pltpu.sync_copy(val_hbm.at[pl.ds(cbase, CHUNKS_PER_PHASE)], valbuf2)

        def off_body(k, _):
            sl = pl.ds(k * LANES, LANES)
            colall[sl] = colall[sl] + col_off
            return 0

        lax.fori_loop(0, EDGES_PER_PHASE // LANES, off_body, 0)

        for k0 in range(AHEAD):
            issue_gather(k0, k0)

        def loop_body(i, _):
            for k in range(NBUF):
                g = i * NBUF + k
                k_nx = (k + AHEAD) % NBUF
                if k < NBUF - AHEAD:
                    @pl.when(i > 0)
                    def _():
                        wait_scatter(k_nx)

                    issue_gather(g + AHEAD, k_nx)
                else:
                    @pl.when(i < NITER - 1)
                    def _():
                        wait_scatter(k_nx)
                        issue_gather(g + AHEAD, k_nx)

                wait_gather(k)
                scale(g, k)
                start_scatter(g, k)
            return 0

        lax.fori_loop(0, NITER, loop_body, 0)
        for k in range(NBUF):
            wait_scatter(k)
        return 0

    lax.fori_loop(0, PHASES, phase_body, 0)
    plsc.subcore_barrier()

    def fin_body(t, _):
        rbase = s * ROWS_PER_TILE + t * SUBSTRIPE
        pltpu.sync_copy(acc.at[pl.ds(rbase, SUBSTRIPE)], outbuf)

        def relu_body(i, _):
            for j in range(HALF // LANES):
                sl = pl.ds(j * LANES, LANES)
                outbuf[i, sl] = jnp.maximum(outbuf[i, sl], 0.0)
            return 0

        lax.fori_loop(0, SUBSTRIPE, relu_body, 0)
        csl = pl.ds(c * HALF, HALF)

        @pl.when(rbase + SUBSTRIPE <= N)
        def _():
            pltpu.sync_copy(outbuf,
                            out_hbm.at[pl.ds(rbase, SUBSTRIPE), csl])

        @pl.when(jnp.logical_and(rbase < N, rbase + SUBSTRIPE > N))
        def _():
            pltpu.sync_copy(
                outbuf.at[pl.ds(0, N - (NS - 1) * ROWS_PER_TILE
                                - (ROWS_PER_TILE // SUBSTRIPE - 1) * SUBSTRIPE)],
                out_hbm.at[pl.ds((NS - 1) * ROWS_PER_TILE
                                 + (ROWS_PER_TILE // SUBSTRIPE - 1) * SUBSTRIPE,
                                 N - (NS - 1) * ROWS_PER_TILE
                                 - (ROWS_PER_TILE // SUBSTRIPE - 1) * SUBSTRIPE),
                           csl])
        return 0

    lax.fori_loop(0, ROWS_PER_TILE // SUBSTRIPE, fin_body, 0)


@functools.partial(
    pl.kernel,
    out_type=jax.ShapeDtypeStruct((N, D_OUT), jnp.float32),
    mesh=plsc.VectorSubcoreMesh(core_axis_name="c", subcore_axis_name="s"),
    scratch_types=[
        pltpu.VMEM((EDGES_PER_PHASE,), jnp.int32),
        pltpu.VMEM((CHUNKS_PER_PHASE, CHUNK), jnp.int32),
        pltpu.VMEM((CHUNKS_PER_PHASE, CHUNK), jnp.float32),
        [pltpu.VMEM((CHUNK, HALF), jnp.float32) for _ in range(NBUF)],
        pltpu.VMEM((SUBSTRIPE, HALF), jnp.float32),
        pltpu.VMEM_SHARED((NPAD, HALF), jnp.float32),
        [pltpu.SemaphoreType.DMA for _ in range(NBUF)],
        [pltpu.SemaphoreType.DMA for _ in range(NBUF)],
    ],
    compiler_params=pltpu.CompilerParams(use_tc_tiling_on_sc=False),
)
def _sc_aggregate(xw_hbm, row_hbm, col_hbm, val_hbm, out_hbm,
                  colall, rowbuf2, valbuf2, rows, outbuf, acc, gsem, ssem):
    _sc_body(xw_hbm, row_hbm, col_hbm, val_hbm, out_hbm,
             colall, rowbuf2, valbuf2, rows, outbuf, acc, gsem, ssem)


@jax.jit
def kernel(x, adj_row, adj_col, adj_val, W):
    W2 = W.reshape(D_IN, 2, HALF).transpose(1, 0, 2)
    xw = _matmul_split(x, W2)
    row2 = adj_row.reshape(E // CHUNK, CHUNK)
    val2 = adj_val.reshape(E // CHUNK, CHUNK)
    return _sc_aggregate(xw, row2, adj_col, val2)
